# baseline (speedup 1.0000x reference)
import jax
import jax.numpy as jnp
from jax.experimental import pallas as pl
from jax.experimental.pallas import tpu as pltpu

B, MAXN, NVT, NET, HS = 256, 10, 16, 8, 100
HP = 128
_F32 = jnp.float32


def _dotT(a, b):
    return jax.lax.dot_general(a, b, (((1,), (1,)), ((), ())),
                               preferred_element_type=_F32)


def _onehot(col, n):
    i = jax.lax.broadcasted_iota(jnp.int32, (col.shape[0], n), 1)
    return (col == i).astype(_F32)


def _sig(x):
    return 0.5 * jnp.tanh(0.5 * x) + 0.5


def _bn(x, g, beta):
    m = jnp.mean(x, axis=0, keepdims=True)
    xc = x - m
    var = jnp.mean(xc * xc, axis=0, keepdims=True)
    return g * xc * jax.lax.rsqrt(var + 1e-5) + beta


def _body(vt, st, stT,
          wi_e_f, wh_e_f, wi_v_f, wh_v_f, gw_f, mw_f,
          wi_e_b, wh_e_b, wi_v_b, wh_v_b, gw_b, mw_b,
          uW, w1r, bc, hr, out_ref,
          hs, ghc, ef, eb, af, ab, t9f, whe3f, gmwf, whv3f, t17f,
          t9b, whe3b, gmwb, whv3b, t17b, uWs, w1s):
    for u in range(MAXN):
        blk = slice(u * B, (u + 1) * B)
        ef[blk, :] = st[0, :, u, :]
        af[blk, :] = st[1, :, u, :].astype(_F32)
        eb[blk, :] = stT[0, :, u, :]
        ab[blk, :] = stT[1, :, u, :].astype(_F32)

    def rowpad(x, width):
        return jnp.concatenate(
            [x, jnp.zeros((1, width - x.shape[1]), _F32)], axis=1)

    def prologue_dir(wi_e, wh_e, wi_v, wh_v, gw, mw, o,
                     t9s, whe3s, gmws, whv3s, t17s):
        t9s[...] = jnp.zeros((5 * HP, NET), _F32)
        t9s[0:HS, :] = wi_e[0:HS, :] + (bc[o:o + HS] + bc[o + 300:o + 400])
        t9s[HP:HP + HS, :] = wi_e[HS:2 * HS, :] + (bc[o + HS:o + 200]
                                                   + bc[o + 400:o + 500])
        t9s[2 * HP:2 * HP + HS, :] = wi_e[2 * HS:3 * HS, :] + bc[o + 200:o + 300]
        t9s[3 * HP:3 * HP + HS, :] = gw[:, HS:] + bc[o + 1000:o + 1100]
        t9s[4 * HP:4 * HP + HS, :] = mw[:, HS:]
        whe3s[...] = jnp.zeros((3 * HP, HP), _F32)
        whe3s[0:HS, 0:HS] = wh_e[0:HS, :]
        whe3s[HP:HP + HS, 0:HS] = wh_e[HS:2 * HS, :]
        whe3s[2 * HP:2 * HP + HS, 0:HS] = wh_e[2 * HS:3 * HS, :]
        gmws[...] = jnp.zeros((2 * HP, HP), _F32)
        gmws[0:HS, 0:HS] = gw[:, 0:HS]
        gmws[HP:HP + HS, 0:HS] = mw[:, 0:HS]
        whv3s[...] = jnp.zeros((3 * HP, HP), _F32)
        whv3s[0:HS, 0:HS] = wh_v[0:HS, :]
        whv3s[HP:HP + HS, 0:HS] = wh_v[HS:2 * HS, :]
        whv3s[2 * HP:2 * HP + HS, 0:HS] = wh_v[2 * HS:3 * HS, :]
        t17s[...] = jnp.zeros((3 * HP, NVT), _F32)
        t17s[0:HS, :] = wi_v[0:HS, :] + (bc[o + 500:o + 600] + bc[o + 800:o + 900])
        t17s[HP:HP + HS, :] = wi_v[HS:2 * HS, :] + (bc[o + 600:o + 700]
                                                    + bc[o + 900:o + 1000])
        t17s[2 * HP:2 * HP + HS, :] = wi_v[2 * HS:3 * HS, :] + bc[o + 700:o + 800]

    prologue_dir(wi_e_f, wh_e_f, wi_v_f, wh_v_f, gw_f, mw_f, 0,
                 t9f, whe3f, gmwf, whv3f, t17f)
    prologue_dir(wi_e_b, wh_e_b, wi_v_b, wh_v_b, gw_b, mw_b, 1100,
                 t9b, whe3b, gmwb, whv3b, t17b)
    uWs[...] = jnp.zeros((HP, 2 * HP), _F32)
    uWs[0:HS, 0:HS] = uW[:, 0:HS]
    uWs[0:HS, HP:HP + HS] = uW[:, HS:2 * HS]
    w1s[...] = jnp.zeros((2 * HP, HP), _F32)
    w1s[0:2 * HS, 0:HS] = w1r[...]
    u_b = rowpad(hr[0:1, 0:HS], HP)
    u_g = rowpad(hr[0:1, HS:2 * HS], HP)
    u_beta = rowpad(hr[0:1, 2 * HS:300], HP)
    c_b1 = rowpad(hr[0:1, 300:500], 2 * HP)
    c_g = rowpad(hr[0:1, 500:700], 2 * HP)
    c_beta = jnp.concatenate(
        [hr[0:1, 700:900], jnp.ones((1, 2 * HP - 200), _F32)], axis=1)
    w2 = jnp.concatenate(
        [hr[0:1, 900:1101], jnp.zeros((1, 2 * HP - 201), _F32)], axis=1)
    bhe_n = {True: rowpad(hr[0:1, 1101:1201], HP),
             False: rowpad(hr[0:1, 1301:1401], HP)}
    bhv_n = {True: rowpad(hr[0:1, 1201:1301], HP),
             False: rowpad(hr[0:1, 1401:1501], HP)}
    wdir = {True: (t9f, whe3f, gmwf, whv3f, t17f),
            False: (t9b, whe3b, gmwb, whv3b, t17b)}

    def run_pass(fwd, H0):
        (t9, whe3, gmw, whv3, t17) = wdir[fwd]
        e_ref = ef if fwd else eb
        a_ref = af if fwd else ab
        order = range(MAXN) if fwd else range(MAXN - 1, -1, -1)
        hv_start = None
        for step, v in enumerate(order):
            if step == 0:
                H = H0
            else:
                lo, hi = (0, v * B) if fwd else ((v + 1) * B, MAXN * B)
                h_nb = hs[lo:hi, :]
                oh8 = _onehot(e_ref[lo:hi, v:v + 1], NET)
                gx = _dotT(oh8, t9[...])
                gh3 = ghc[lo:hi, :]
                r = _sig(gx[:, 0:HP] + gh3[:, 0:HP])
                z = _sig(gx[:, HP:2 * HP] + gh3[:, HP:2 * HP])
                n = jnp.tanh(gx[:, 2 * HP:3 * HP]
                             + r * (gh3[:, 2 * HP:3 * HP] + bhe_n[fwd]))
                He = n + z * (h_nb - n)
                gm = _dotT(He, gmw[...])
                g = _sig(gm[:, 0:HP] + gx[:, 3 * HP:4 * HP])
                mp = gm[:, HP:2 * HP] + gx[:, 4 * HP:5 * HP]
                gated = g * mp * a_ref[lo:hi, v:v + 1]
                H = gated[0:B, :]
                for u in range(1, (hi - lo) // B):
                    H = H + gated[u * B:(u + 1) * B, :]
            oh16 = _onehot(vt[:, v:v + 1], NVT)
            gxv = _dotT(oh16, t17[...])
            if H is None:
                r = _sig(gxv[:, 0:HP])
                z = _sig(gxv[:, HP:2 * HP])
                n = jnp.tanh(gxv[:, 2 * HP:3 * HP] + r * bhv_n[fwd])
                Hv = n - z * n
            else:
                ghv = _dotT(H, whv3[...])
                r = _sig(gxv[:, 0:HP] + ghv[:, 0:HP])
                z = _sig(gxv[:, HP:2 * HP] + ghv[:, HP:2 * HP])
                n = jnp.tanh(gxv[:, 2 * HP:3 * HP]
                             + r * (ghv[:, 2 * HP:3 * HP] + bhv_n[fwd]))
                Hv = n + z * (H - n)
            hs[v * B:(v + 1) * B, :] = Hv
            if step < MAXN - 1:
                ghc[v * B:(v + 1) * B, :] = _dotT(Hv, whe3[...])
            if step == 0:
                hv_start = Hv
        return hv_start

    hvf = run_pass(True, None)
    run_pass(True, hvf)
    hf9 = hs[(MAXN - 1) * B:MAXN * B, :]
    hvb = run_pass(False, None)
    run_pass(False, hvb)
    hb0 = hs[0:B, :]

    xin = jnp.concatenate([hf9, hb0], axis=1)
    x = _dotT(xin, uWs[...]) + u_b
    x = _bn(x, u_g, u_beta)
    h1 = jax.nn.relu(_dotT(x, w1s[...]) + c_b1)
    h1 = _bn(h1, c_g, c_beta)
    out_ref[...] = _dotT(h1, w2)


def kernel(v_types, adj, e_types, params):
    p = params
    vt = v_types.astype(jnp.int32)
    st = jnp.stack([e_types.astype(jnp.int32), adj.astype(jnp.int32)])
    stT = jnp.transpose(st, (0, 1, 3, 2))
    bc = jnp.concatenate(
        [x for pre in ('f', 'b') for x in
         (p['grue_' + pre + '_bi'], p['grue_' + pre + '_bh'][:2 * HS],
          p['gruv_' + pre + '_bi'], p['gruv_' + pre + '_bh'][:2 * HS],
          p['gate_' + pre + '_b'])])[:, None]
    hr = jnp.concatenate(
        [p['unify_b'], p['unify_g'], p['unify_beta'], p['cls_b1'],
         p['cls_g'], p['cls_beta'], p['cls_W2'][0], p['cls_b2'],
         p['grue_f_bh'][2 * HS:], p['gruv_f_bh'][2 * HS:],
         p['grue_b_bh'][2 * HS:], p['gruv_b_bh'][2 * HS:]])[None]
    sv = pltpu.VMEM
    return pl.pallas_call(
        _body,
        out_shape=jax.ShapeDtypeStruct((B, 1), _F32),
        scratch_shapes=[sv((MAXN * B, HP), _F32),
                        sv((MAXN * B, 3 * HP), _F32),
                        sv((MAXN * B, MAXN), jnp.int32),
                        sv((MAXN * B, MAXN), jnp.int32),
                        sv((MAXN * B, MAXN), _F32),
                        sv((MAXN * B, MAXN), _F32)] + 2 * [
            sv((5 * HP, NET), _F32), sv((3 * HP, HP), _F32),
            sv((2 * HP, HP), _F32), sv((3 * HP, HP), _F32),
            sv((3 * HP, NVT), _F32)] + [
            sv((HP, 2 * HP), _F32), sv((2 * HP, HP), _F32)],
    )(vt, st, stT,
      p['grue_f_Wi'], p['grue_f_Wh'], p['gruv_f_Wi'], p['gruv_f_Wh'],
      p['gate_f_W'], p['map_f_W'],
      p['grue_b_Wi'], p['grue_b_Wh'], p['gruv_b_Wi'], p['gruv_b_Wh'],
      p['gate_b_W'], p['map_b_W'],
      p['unify_W'], p['cls_W1'], bc, hr)


# speedup vs baseline: 1.6008x; 1.6008x over previous
import jax
import jax.numpy as jnp
from jax.experimental import pallas as pl
from jax.experimental.pallas import tpu as pltpu

B, MAXN, NVT, NET, HS = 256, 10, 16, 8, 100
HP = 128
_F32 = jnp.float32


def _dotT(a, b):
    return jax.lax.dot_general(a, b, (((1,), (1,)), ((), ())),
                               preferred_element_type=_F32)


def _onehot(col, n):
    i = jax.lax.broadcasted_iota(jnp.int32, (col.shape[0], n), 1)
    return (col == i).astype(_F32)


def _sig(x):
    return 0.5 * jnp.tanh(0.5 * x) + 0.5


def _bn(x, g, beta):
    m = jnp.mean(x, axis=0, keepdims=True)
    xc = x - m
    var = jnp.mean(xc * xc, axis=0, keepdims=True)
    return g * xc * jax.lax.rsqrt(var + 1e-5) + beta


def _body(vt, ef, eb, af, ab,
          wi_e_f, wh_e_f, wi_v_f, wh_v_f, gw_f, mw_f,
          wi_e_b, wh_e_b, wi_v_b, wh_v_b, gw_b, mw_b,
          uW, w1r, bc, hr, out_ref,
          hs, ghc, t9f, whe3f, gmwf, whv3f, t17f,
          t9b, whe3b, gmwb, whv3b, t17b, uWs, w1s):

    def rowpad(x, width):
        return jnp.concatenate(
            [x, jnp.zeros((1, width - x.shape[1]), _F32)], axis=1)

    def prologue_dir(wi_e, wh_e, wi_v, wh_v, gw, mw, o,
                     t9s, whe3s, gmws, whv3s, t17s):
        t9s[...] = jnp.zeros((5 * HP, NET), _F32)
        t9s[0:HS, :] = wi_e[0:HS, :] + (bc[o:o + HS] + bc[o + 300:o + 400])
        t9s[HP:HP + HS, :] = wi_e[HS:2 * HS, :] + (bc[o + HS:o + 200]
                                                   + bc[o + 400:o + 500])
        t9s[2 * HP:2 * HP + HS, :] = wi_e[2 * HS:3 * HS, :] + bc[o + 200:o + 300]
        t9s[3 * HP:3 * HP + HS, :] = gw[:, HS:] + bc[o + 1000:o + 1100]
        t9s[4 * HP:4 * HP + HS, :] = mw[:, HS:]
        whe3s[...] = jnp.zeros((3 * HP, HP), _F32)
        whe3s[0:HS, 0:HS] = wh_e[0:HS, :]
        whe3s[HP:HP + HS, 0:HS] = wh_e[HS:2 * HS, :]
        whe3s[2 * HP:2 * HP + HS, 0:HS] = wh_e[2 * HS:3 * HS, :]
        gmws[...] = jnp.zeros((2 * HP, HP), _F32)
        gmws[0:HS, 0:HS] = gw[:, 0:HS]
        gmws[HP:HP + HS, 0:HS] = mw[:, 0:HS]
        whv3s[...] = jnp.zeros((3 * HP, HP), _F32)
        whv3s[0:HS, 0:HS] = wh_v[0:HS, :]
        whv3s[HP:HP + HS, 0:HS] = wh_v[HS:2 * HS, :]
        whv3s[2 * HP:2 * HP + HS, 0:HS] = wh_v[2 * HS:3 * HS, :]
        t17s[...] = jnp.zeros((3 * HP, NVT), _F32)
        t17s[0:HS, :] = wi_v[0:HS, :] + (bc[o + 500:o + 600] + bc[o + 800:o + 900])
        t17s[HP:HP + HS, :] = wi_v[HS:2 * HS, :] + (bc[o + 600:o + 700]
                                                    + bc[o + 900:o + 1000])
        t17s[2 * HP:2 * HP + HS, :] = wi_v[2 * HS:3 * HS, :] + bc[o + 700:o + 800]

    prologue_dir(wi_e_f, wh_e_f, wi_v_f, wh_v_f, gw_f, mw_f, 0,
                 t9f, whe3f, gmwf, whv3f, t17f)
    prologue_dir(wi_e_b, wh_e_b, wi_v_b, wh_v_b, gw_b, mw_b, 1100,
                 t9b, whe3b, gmwb, whv3b, t17b)
    uWs[...] = jnp.zeros((HP, 2 * HP), _F32)
    uWs[0:HS, 0:HS] = uW[:, 0:HS]
    uWs[0:HS, HP:HP + HS] = uW[:, HS:2 * HS]
    w1s[...] = jnp.zeros((2 * HP, HP), _F32)
    w1s[0:2 * HS, 0:HS] = w1r[...]
    u_b = rowpad(hr[0:1, 0:HS], HP)
    u_g = rowpad(hr[0:1, HS:2 * HS], HP)
    u_beta = rowpad(hr[0:1, 2 * HS:300], HP)
    c_b1 = rowpad(hr[0:1, 300:500], 2 * HP)
    c_g = rowpad(hr[0:1, 500:700], 2 * HP)
    c_beta = jnp.concatenate(
        [hr[0:1, 700:900], jnp.ones((1, 2 * HP - 200), _F32)], axis=1)
    w2 = jnp.concatenate(
        [hr[0:1, 900:1101], jnp.zeros((1, 2 * HP - 201), _F32)], axis=1)
    bhe_n = {True: rowpad(hr[0:1, 1101:1201], HP),
             False: rowpad(hr[0:1, 1301:1401], HP)}
    bhv_n = {True: rowpad(hr[0:1, 1201:1301], HP),
             False: rowpad(hr[0:1, 1401:1501], HP)}
    wdir = {True: (t9f, whe3f, gmwf, whv3f, t17f),
            False: (t9b, whe3b, gmwb, whv3b, t17b)}

    def run_pass(fwd, H0):
        (t9, whe3, gmw, whv3, t17) = wdir[fwd]
        e_ref = ef if fwd else eb
        a_ref = af if fwd else ab
        order = range(MAXN) if fwd else range(MAXN - 1, -1, -1)
        hv_start = None
        for step, v in enumerate(order):
            if step == 0:
                H = H0
            else:
                lo, hi = (0, v * B) if fwd else ((v + 1) * B, MAXN * B)
                h_nb = hs[lo:hi, :]
                oh8 = _onehot(e_ref[lo:hi, v:v + 1], NET)
                gx = _dotT(oh8, t9[...])
                gh3 = ghc[lo:hi, :]
                r = _sig(gx[:, 0:HP] + gh3[:, 0:HP])
                z = _sig(gx[:, HP:2 * HP] + gh3[:, HP:2 * HP])
                n = jnp.tanh(gx[:, 2 * HP:3 * HP]
                             + r * (gh3[:, 2 * HP:3 * HP] + bhe_n[fwd]))
                He = n + z * (h_nb - n)
                gm = _dotT(He, gmw[...])
                g = _sig(gm[:, 0:HP] + gx[:, 3 * HP:4 * HP])
                mp = gm[:, HP:2 * HP] + gx[:, 4 * HP:5 * HP]
                gated = g * mp * a_ref[lo:hi, v:v + 1]
                H = gated[0:B, :]
                for u in range(1, (hi - lo) // B):
                    H = H + gated[u * B:(u + 1) * B, :]
            oh16 = _onehot(vt[:, v:v + 1], NVT)
            gxv = _dotT(oh16, t17[...])
            if H is None:
                r = _sig(gxv[:, 0:HP])
                z = _sig(gxv[:, HP:2 * HP])
                n = jnp.tanh(gxv[:, 2 * HP:3 * HP] + r * bhv_n[fwd])
                Hv = n - z * n
            else:
                ghv = _dotT(H, whv3[...])
                r = _sig(gxv[:, 0:HP] + ghv[:, 0:HP])
                z = _sig(gxv[:, HP:2 * HP] + ghv[:, HP:2 * HP])
                n = jnp.tanh(gxv[:, 2 * HP:3 * HP]
                             + r * (ghv[:, 2 * HP:3 * HP] + bhv_n[fwd]))
                Hv = n + z * (H - n)
            hs[v * B:(v + 1) * B, :] = Hv
            if step < MAXN - 1:
                ghc[v * B:(v + 1) * B, :] = _dotT(Hv, whe3[...])
            if step == 0:
                hv_start = Hv
        return hv_start

    def first_vertex(fwd):
        v = 0 if fwd else MAXN - 1
        oh16 = _onehot(vt[:, v:v + 1], NVT)
        gxv = _dotT(oh16, wdir[fwd][4][...])
        r = _sig(gxv[:, 0:HP])
        z = _sig(gxv[:, HP:2 * HP])
        n = jnp.tanh(gxv[:, 2 * HP:3 * HP] + r * bhv_n[fwd])
        return n - z * n

    run_pass(True, first_vertex(True))
    hf9 = hs[(MAXN - 1) * B:MAXN * B, :]
    run_pass(False, first_vertex(False))
    hb0 = hs[0:B, :]

    xin = jnp.concatenate([hf9, hb0], axis=1)
    x = _dotT(xin, uWs[...]) + u_b
    x = _bn(x, u_g, u_beta)
    h1 = jax.nn.relu(_dotT(x, w1s[...]) + c_b1)
    h1 = _bn(h1, c_g, c_beta)
    out_ref[...] = _dotT(h1, w2)


def kernel(v_types, adj, e_types, params):
    p = params
    vt = v_types.astype(jnp.int32)
    st = jnp.stack([e_types.astype(jnp.int32), adj.astype(jnp.int32)])
    tf = jnp.transpose(st, (0, 2, 1, 3)).reshape(2, MAXN * B, MAXN)
    tb = jnp.transpose(st, (0, 3, 1, 2)).reshape(2, MAXN * B, MAXN)
    ef, af = tf[0], tf[1].astype(_F32)
    eb, ab = tb[0], tb[1].astype(_F32)
    bc = jnp.concatenate(
        [x for pre in ('f', 'b') for x in
         (p['grue_' + pre + '_bi'], p['grue_' + pre + '_bh'][:2 * HS],
          p['gruv_' + pre + '_bi'], p['gruv_' + pre + '_bh'][:2 * HS],
          p['gate_' + pre + '_b'])])[:, None]
    hr = jnp.concatenate(
        [p['unify_b'], p['unify_g'], p['unify_beta'], p['cls_b1'],
         p['cls_g'], p['cls_beta'], p['cls_W2'][0], p['cls_b2'],
         p['grue_f_bh'][2 * HS:], p['gruv_f_bh'][2 * HS:],
         p['grue_b_bh'][2 * HS:], p['gruv_b_bh'][2 * HS:]])[None]
    sv = pltpu.VMEM
    return pl.pallas_call(
        _body,
        out_shape=jax.ShapeDtypeStruct((B, 1), _F32),
        scratch_shapes=[sv((MAXN * B, HP), _F32),
                        sv((MAXN * B, 3 * HP), _F32)] + 2 * [
            sv((5 * HP, NET), _F32), sv((3 * HP, HP), _F32),
            sv((2 * HP, HP), _F32), sv((3 * HP, HP), _F32),
            sv((3 * HP, NVT), _F32)] + [
            sv((HP, 2 * HP), _F32), sv((2 * HP, HP), _F32)],
    )(vt, ef, eb, af, ab,
      p['grue_f_Wi'], p['grue_f_Wh'], p['gruv_f_Wi'], p['gruv_f_Wh'],
      p['gate_f_W'], p['map_f_W'],
      p['grue_b_Wi'], p['grue_b_Wh'], p['gruv_b_Wi'], p['gruv_b_Wh'],
      p['gate_b_W'], p['map_b_W'],
      p['unify_W'], p['cls_W1'], bc, hr)


# speedup vs baseline: 1.6702x; 1.0434x over previous
import jax
import jax.numpy as jnp
from jax.experimental import pallas as pl
from jax.experimental.pallas import tpu as pltpu

B, MAXN, NVT, NET, HS = 256, 10, 16, 8, 100
HP = 128
_F32 = jnp.float32


def _dotT(a, b):
    return jax.lax.dot_general(a, b, (((1,), (1,)), ((), ())),
                               preferred_element_type=_F32)


def _onehot(col, n):
    i = jax.lax.broadcasted_iota(jnp.int32, (col.shape[0], n), 1)
    return (col == i).astype(_F32)


def _sig(x):
    return 0.5 * jnp.tanh(0.5 * x) + 0.5


def _bn(x, g, beta):
    m = jnp.mean(x, axis=0, keepdims=True)
    xc = x - m
    var = jnp.mean(xc * xc, axis=0, keepdims=True)
    return g * xc * jax.lax.rsqrt(var + 1e-5) + beta


def _body(vt, tfr, tbr,
          wi_e_f, wh_e_f, wi_v_f, wh_v_f, gw_f, mw_f,
          wi_e_b, wh_e_b, wi_v_b, wh_v_b, gw_b, mw_b,
          uW, w1r, bc, hr, out_ref,
          hs, ghc, t9f, whe3f, gmwf, whv3f, t17f,
          t9b, whe3b, gmwb, whv3b, t17b, uWs, w1s):

    def rowpad(x, width):
        return jnp.concatenate(
            [x, jnp.zeros((1, width - x.shape[1]), _F32)], axis=1)

    def prologue_dir(wi_e, wh_e, wi_v, wh_v, gw, mw, o,
                     t9s, whe3s, gmws, whv3s, t17s):
        t9s[...] = jnp.zeros((5 * HP, NET), _F32)
        t9s[0:HS, :] = wi_e[0:HS, :] + (bc[o:o + HS] + bc[o + 300:o + 400])
        t9s[HP:HP + HS, :] = wi_e[HS:2 * HS, :] + (bc[o + HS:o + 200]
                                                   + bc[o + 400:o + 500])
        t9s[2 * HP:2 * HP + HS, :] = wi_e[2 * HS:3 * HS, :] + bc[o + 200:o + 300]
        t9s[3 * HP:3 * HP + HS, :] = gw[:, HS:] + bc[o + 1000:o + 1100]
        t9s[4 * HP:4 * HP + HS, :] = mw[:, HS:]
        whe3s[...] = jnp.zeros((3 * HP, HP), _F32)
        whe3s[0:HS, 0:HS] = wh_e[0:HS, :]
        whe3s[HP:HP + HS, 0:HS] = wh_e[HS:2 * HS, :]
        whe3s[2 * HP:2 * HP + HS, 0:HS] = wh_e[2 * HS:3 * HS, :]
        gmws[...] = jnp.zeros((2 * HP, HP), _F32)
        gmws[0:HS, 0:HS] = gw[:, 0:HS]
        gmws[HP:HP + HS, 0:HS] = mw[:, 0:HS]
        whv3s[...] = jnp.zeros((3 * HP, HP), _F32)
        whv3s[0:HS, 0:HS] = wh_v[0:HS, :]
        whv3s[HP:HP + HS, 0:HS] = wh_v[HS:2 * HS, :]
        whv3s[2 * HP:2 * HP + HS, 0:HS] = wh_v[2 * HS:3 * HS, :]
        t17s[...] = jnp.zeros((3 * HP, NVT), _F32)
        t17s[0:HS, :] = wi_v[0:HS, :] + (bc[o + 500:o + 600] + bc[o + 800:o + 900])
        t17s[HP:HP + HS, :] = wi_v[HS:2 * HS, :] + (bc[o + 600:o + 700]
                                                    + bc[o + 900:o + 1000])
        t17s[2 * HP:2 * HP + HS, :] = wi_v[2 * HS:3 * HS, :] + bc[o + 700:o + 800]

    prologue_dir(wi_e_f, wh_e_f, wi_v_f, wh_v_f, gw_f, mw_f, 0,
                 t9f, whe3f, gmwf, whv3f, t17f)
    prologue_dir(wi_e_b, wh_e_b, wi_v_b, wh_v_b, gw_b, mw_b, 1100,
                 t9b, whe3b, gmwb, whv3b, t17b)
    uWs[...] = jnp.zeros((HP, 2 * HP), _F32)
    uWs[0:HS, 0:HS] = uW[:, 0:HS]
    uWs[0:HS, HP:HP + HS] = uW[:, HS:2 * HS]
    w1s[...] = jnp.zeros((2 * HP, HP), _F32)
    w1s[0:2 * HS, 0:HS] = w1r[...]
    u_b = rowpad(hr[0:1, 0:HS], HP)
    u_g = rowpad(hr[0:1, HS:2 * HS], HP)
    u_beta = rowpad(hr[0:1, 2 * HS:300], HP)
    c_b1 = rowpad(hr[0:1, 300:500], 2 * HP)
    c_g = rowpad(hr[0:1, 500:700], 2 * HP)
    c_beta = jnp.concatenate(
        [hr[0:1, 700:900], jnp.ones((1, 2 * HP - 200), _F32)], axis=1)
    w2 = jnp.concatenate(
        [hr[0:1, 900:1101], jnp.zeros((1, 2 * HP - 201), _F32)], axis=1)
    bhe_n = {True: rowpad(hr[0:1, 1101:1201], HP),
             False: rowpad(hr[0:1, 1301:1401], HP)}
    bhv_n = {True: rowpad(hr[0:1, 1201:1301], HP),
             False: rowpad(hr[0:1, 1401:1501], HP)}
    wdir = {True: (t9f, whe3f, gmwf, whv3f, t17f),
            False: (t9b, whe3b, gmwb, whv3b, t17b)}

    def run_pass(fwd, H0):
        (t9, whe3, gmw, whv3, t17) = wdir[fwd]
        ea_ref = tfr if fwd else tbr
        order = range(MAXN) if fwd else range(MAXN - 1, -1, -1)
        hv_start = None
        for step, v in enumerate(order):
            if step == 0:
                H = H0
            else:
                lo, hi = (0, v * B) if fwd else ((v + 1) * B, MAXN * B)
                h_nb = hs[lo:hi, :]
                oh8 = _onehot(ea_ref[0, lo:hi, v:v + 1], NET)
                gx = _dotT(oh8, t9[...])
                gh3 = ghc[lo:hi, :]
                r = _sig(gx[:, 0:HP] + gh3[:, 0:HP])
                z = _sig(gx[:, HP:2 * HP] + gh3[:, HP:2 * HP])
                n = jnp.tanh(gx[:, 2 * HP:3 * HP]
                             + r * (gh3[:, 2 * HP:3 * HP] + bhe_n[fwd]))
                He = n + z * (h_nb - n)
                gm = _dotT(He, gmw[...])
                g = _sig(gm[:, 0:HP] + gx[:, 3 * HP:4 * HP])
                mp = gm[:, HP:2 * HP] + gx[:, 4 * HP:5 * HP]
                gated = g * mp * ea_ref[1, lo:hi, v:v + 1].astype(_F32)
                H = gated[0:B, :]
                for u in range(1, (hi - lo) // B):
                    H = H + gated[u * B:(u + 1) * B, :]
            oh16 = _onehot(vt[:, v:v + 1], NVT)
            gxv = _dotT(oh16, t17[...])
            if H is None:
                r = _sig(gxv[:, 0:HP])
                z = _sig(gxv[:, HP:2 * HP])
                n = jnp.tanh(gxv[:, 2 * HP:3 * HP] + r * bhv_n[fwd])
                Hv = n - z * n
            else:
                ghv = _dotT(H, whv3[...])
                r = _sig(gxv[:, 0:HP] + ghv[:, 0:HP])
                z = _sig(gxv[:, HP:2 * HP] + ghv[:, HP:2 * HP])
                n = jnp.tanh(gxv[:, 2 * HP:3 * HP]
                             + r * (ghv[:, 2 * HP:3 * HP] + bhv_n[fwd]))
                Hv = n + z * (H - n)
            hs[v * B:(v + 1) * B, :] = Hv
            if step < MAXN - 1:
                ghc[v * B:(v + 1) * B, :] = _dotT(Hv, whe3[...])
            if step == 0:
                hv_start = Hv
        return hv_start

    def first_vertex(fwd):
        v = 0 if fwd else MAXN - 1
        oh16 = _onehot(vt[:, v:v + 1], NVT)
        gxv = _dotT(oh16, wdir[fwd][4][...])
        r = _sig(gxv[:, 0:HP])
        z = _sig(gxv[:, HP:2 * HP])
        n = jnp.tanh(gxv[:, 2 * HP:3 * HP] + r * bhv_n[fwd])
        return n - z * n

    run_pass(True, first_vertex(True))
    hf9 = hs[(MAXN - 1) * B:MAXN * B, :]
    run_pass(False, first_vertex(False))
    hb0 = hs[0:B, :]

    xin = jnp.concatenate([hf9, hb0], axis=1)
    x = _dotT(xin, uWs[...]) + u_b
    x = _bn(x, u_g, u_beta)
    h1 = jax.nn.relu(_dotT(x, w1s[...]) + c_b1)
    h1 = _bn(h1, c_g, c_beta)
    out_ref[...] = _dotT(h1, w2)


def kernel(v_types, adj, e_types, params):
    p = params
    vt = v_types.astype(jnp.int32)
    st = jnp.stack([e_types.astype(jnp.int32), adj.astype(jnp.int32)])
    tf = jnp.transpose(st, (0, 2, 1, 3)).reshape(2, MAXN * B, MAXN)
    tb = jnp.transpose(st, (0, 3, 1, 2)).reshape(2, MAXN * B, MAXN)
    bc = jnp.concatenate(
        [x for pre in ('f', 'b') for x in
         (p['grue_' + pre + '_bi'], p['grue_' + pre + '_bh'][:2 * HS],
          p['gruv_' + pre + '_bi'], p['gruv_' + pre + '_bh'][:2 * HS],
          p['gate_' + pre + '_b'])])[:, None]
    hr = jnp.concatenate(
        [p['unify_b'], p['unify_g'], p['unify_beta'], p['cls_b1'],
         p['cls_g'], p['cls_beta'], p['cls_W2'][0], p['cls_b2'],
         p['grue_f_bh'][2 * HS:], p['gruv_f_bh'][2 * HS:],
         p['grue_b_bh'][2 * HS:], p['gruv_b_bh'][2 * HS:]])[None]
    sv = pltpu.VMEM
    return pl.pallas_call(
        _body,
        out_shape=jax.ShapeDtypeStruct((B, 1), _F32),
        scratch_shapes=[sv((MAXN * B, HP), _F32),
                        sv((MAXN * B, 3 * HP), _F32)] + 2 * [
            sv((5 * HP, NET), _F32), sv((3 * HP, HP), _F32),
            sv((2 * HP, HP), _F32), sv((3 * HP, HP), _F32),
            sv((3 * HP, NVT), _F32)] + [
            sv((HP, 2 * HP), _F32), sv((2 * HP, HP), _F32)],
    )(vt, tf, tb,
      p['grue_f_Wi'], p['grue_f_Wh'], p['gruv_f_Wi'], p['gruv_f_Wh'],
      p['gate_f_W'], p['map_f_W'],
      p['grue_b_Wi'], p['grue_b_Wh'], p['gruv_b_Wi'], p['gruv_b_Wh'],
      p['gate_b_W'], p['map_b_W'],
      p['unify_W'], p['cls_W1'], bc, hr)
